# Initial kernel scaffold; baseline (speedup 1.0000x reference)
import jax
import jax.numpy as jnp
from jax.experimental import pallas as pl


def kernel(batch_source, batch_r, entity_embeddings, relation_embeddings):
    raise NotImplementedError("write your pallas kernel here")


# speedup vs baseline: 2.4065x; 2.4065x over previous
import dataclasses
import functools

import jax
import jax.numpy as jnp
from jax import lax
from jax.experimental import pallas as pl
from jax.experimental.pallas import tpu as pltpu
from jax.experimental.pallas import tpu_sc as plsc

LANES = 16
NUM_WORKERS = 32
CHUNK = 128


def _vrsqrt(s):
    i = lax.bitcast_convert_type(s, jnp.int32)
    i = jnp.int32(0x5F3759DF) - (i >> 1)
    y = lax.bitcast_convert_type(i, jnp.float32)
    h = s * 0.5
    for _ in range(3):
        y = y * (1.5 - h * y * y)
    return y


def _transe_body(nch, ent_hbm, rel_hbm, idxe_hbm, idxr_hbm, out_hbm,
                 idxe_v, idxr_v, bufe, bufr, bufo):
    d = ent_hbm.shape[1]
    nvec = d // LANES
    wid = lax.axis_index("s") * 2 + lax.axis_index("c")
    base = wid * (nch * CHUNK)

    pltpu.sync_copy(idxe_hbm.at[wid], idxe_v)
    pltpu.sync_copy(idxr_hbm.at[wid], idxr_v)

    for j in range(nch):
        pltpu.sync_copy(ent_hbm.at[idxe_v.at[j]], bufe)
        pltpu.sync_copy(rel_hbm.at[idxr_v.at[j]], bufr)

        @pl.loop(0, CHUNK)
        def _(r):
            evs = [bufe[r, pl.ds(k * LANES, LANES)] for k in range(nvec)]
            rvs = [bufr[r, pl.ds(k * LANES, LANES)] for k in range(nvec)]
            acc_e = evs[0] * evs[0]
            acc_r = rvs[0] * rvs[0]
            for k in range(1, nvec):
                acc_e = acc_e + evs[k] * evs[k]
                acc_r = acc_r + rvs[k] * rvs[k]
            se = jnp.maximum(jnp.sum(acc_e), 1e-12)
            sr = jnp.maximum(jnp.sum(acc_r), 1e-12)
            ye = _vrsqrt(jnp.broadcast_to(se, (LANES,)))
            yr = _vrsqrt(jnp.broadcast_to(sr, (LANES,)))
            for k in range(nvec):
                bufo[r, pl.ds(k * LANES, LANES)] = evs[k] * ye + rvs[k] * yr

        pltpu.sync_copy(bufo, out_hbm.at[pl.ds(base + j * CHUNK, CHUNK)])


def kernel(batch_source, batch_r, entity_embeddings, relation_embeddings):
    b = batch_source.shape[0]
    d = entity_embeddings.shape[1]
    nch = b // (NUM_WORKERS * CHUNK)
    idx_e = batch_source.astype(jnp.int32).reshape(NUM_WORKERS, nch, CHUNK)
    idx_r = batch_r.astype(jnp.int32).reshape(NUM_WORKERS, nch, CHUNK)

    mesh = plsc.VectorSubcoreMesh(core_axis_name="c", subcore_axis_name="s")
    cp = pltpu.CompilerParams()
    if "needs_layout_passes" in pltpu.CompilerParams.__dataclass_fields__:
        cp = dataclasses.replace(cp, needs_layout_passes=False)
    run = pl.kernel(
        functools.partial(_transe_body, nch),
        out_type=jax.ShapeDtypeStruct((b, d), jnp.float32),
        mesh=mesh,
        scratch_types=[
            pltpu.VMEM((nch, CHUNK), jnp.int32),
            pltpu.VMEM((nch, CHUNK), jnp.int32),
            pltpu.VMEM((CHUNK, d), jnp.float32),
            pltpu.VMEM((CHUNK, d), jnp.float32),
            pltpu.VMEM((CHUNK, d), jnp.float32),
        ],
        compiler_params=cp,
    )
    return run(entity_embeddings, relation_embeddings, idx_e, idx_r)


# speedup vs baseline: 2.9583x; 1.2293x over previous
import dataclasses
import functools

import jax
import jax.numpy as jnp
from jax import lax
from jax.experimental import pallas as pl
from jax.experimental.pallas import tpu as pltpu
from jax.experimental.pallas import tpu_sc as plsc

LANES = 16
NUM_WORKERS = 32
CHUNK = 128


def _vrsqrt(s):
    i = lax.bitcast_convert_type(s, jnp.int32)
    i = jnp.int32(0x5F3759DF) - (i >> 1)
    y = lax.bitcast_convert_type(i, jnp.float32)
    h = s * 0.5
    for _ in range(3):
        y = y * (1.5 - h * y * y)
    return y


def _transe_body(nch, ent_hbm, rel_hbm, idxe_hbm, idxr_hbm, out_hbm,
                 idxe_v, idxr_v, bufe0, bufe1, bufr0, bufr1, bufo0, bufo1,
                 seme0, seme1, semr0, semr1, semo0, semo1):
    d = ent_hbm.shape[1]
    nvec = d // LANES
    wid = lax.axis_index("s") * 2 + lax.axis_index("c")
    base = wid * (nch * CHUNK)

    bufe = [bufe0, bufe1]
    bufr = [bufr0, bufr1]
    bufo = [bufo0, bufo1]
    seme = [seme0, seme1]
    semr = [semr0, semr1]
    semo = [semo0, semo1]

    pltpu.sync_copy(idxe_hbm.at[wid], idxe_v)
    pltpu.sync_copy(idxr_hbm.at[wid], idxr_v)

    gath = [None, None]
    outcp = [None, None]
    gath[0] = (
        pltpu.async_copy(ent_hbm.at[idxe_v.at[0]], bufe[0], seme[0]),
        pltpu.async_copy(rel_hbm.at[idxr_v.at[0]], bufr[0], semr[0]),
    )
    for j in range(nch):
        cur = j % 2
        nxt = (j + 1) % 2
        if j + 1 < nch:
            gath[nxt] = (
                pltpu.async_copy(ent_hbm.at[idxe_v.at[j + 1]], bufe[nxt],
                                 seme[nxt]),
                pltpu.async_copy(rel_hbm.at[idxr_v.at[j + 1]], bufr[nxt],
                                 semr[nxt]),
            )
        gath[cur][0].wait()
        gath[cur][1].wait()
        if outcp[cur] is not None:
            outcp[cur].wait()
        be, br, bo = bufe[cur], bufr[cur], bufo[cur]

        @pl.loop(0, CHUNK)
        def _(r):
            evs = [be[r, pl.ds(k * LANES, LANES)] for k in range(nvec)]
            rvs = [br[r, pl.ds(k * LANES, LANES)] for k in range(nvec)]
            acc_e = evs[0] * evs[0]
            acc_r = rvs[0] * rvs[0]
            for k in range(1, nvec):
                acc_e = acc_e + evs[k] * evs[k]
                acc_r = acc_r + rvs[k] * rvs[k]
            se = jnp.maximum(jnp.sum(acc_e), 1e-12)
            sr = jnp.maximum(jnp.sum(acc_r), 1e-12)
            ye = _vrsqrt(jnp.broadcast_to(se, (LANES,)))
            yr = _vrsqrt(jnp.broadcast_to(sr, (LANES,)))
            for k in range(nvec):
                bo[r, pl.ds(k * LANES, LANES)] = evs[k] * ye + rvs[k] * yr

        outcp[cur] = pltpu.async_copy(
            bo, out_hbm.at[pl.ds(base + j * CHUNK, CHUNK)], semo[cur])

    for cp in outcp:
        if cp is not None:
            cp.wait()


def kernel(batch_source, batch_r, entity_embeddings, relation_embeddings):
    b = batch_source.shape[0]
    d = entity_embeddings.shape[1]
    nch = b // (NUM_WORKERS * CHUNK)
    idx_e = batch_source.astype(jnp.int32).reshape(NUM_WORKERS, nch, CHUNK)
    idx_r = batch_r.astype(jnp.int32).reshape(NUM_WORKERS, nch, CHUNK)

    mesh = plsc.VectorSubcoreMesh(core_axis_name="c", subcore_axis_name="s")
    cp = pltpu.CompilerParams()
    if "needs_layout_passes" in pltpu.CompilerParams.__dataclass_fields__:
        cp = dataclasses.replace(cp, needs_layout_passes=False)
    run = pl.kernel(
        functools.partial(_transe_body, nch),
        out_type=jax.ShapeDtypeStruct((b, d), jnp.float32),
        mesh=mesh,
        scratch_types=(
            [pltpu.VMEM((nch, CHUNK), jnp.int32)] * 2
            + [pltpu.VMEM((CHUNK, d), jnp.float32)] * 6
            + [pltpu.SemaphoreType.DMA] * 6
        ),
        compiler_params=cp,
    )
    return run(entity_embeddings, relation_embeddings, idx_e, idx_r)
